# baseline (speedup 1.0000x reference)
import functools

import numpy as np
import jax
import jax.numpy as jnp
from jax import lax
from jax.experimental import pallas as pl
from jax.experimental.pallas import tpu as pltpu
from jax.experimental.pallas import tpu_sc as plsc

N = 10000
E = 320000
INPUT_DIM = 64
HID_DIM = 64
OUTPUT_DIM = 64
B = 8
INPUT_SIZE = INPUT_DIM + HID_DIM
NUM_MATRICES = 5
C = INPUT_SIZE * B

NW = 32
RPW = 320
NPAD = NW * RPW
NCH = 4
FC = C // NCH
GK = 64
LANES = 16

_idx = np.arange(INPUT_SIZE * NUM_MATRICES)
_PERM = np.asarray((_idx % INPUT_SIZE) * NUM_MATRICES + _idx // INPUT_SIZE,
                   dtype=np.int32)


def _spmm_body(has_prev, alpha, *refs):
    if has_prev:
        (rows, cols, vals, est, tab, xprev, out,
         acc, gbuf, colv, colv2, rowv, valv, esv, gsem, msem) = refs
    else:
        (rows, cols, vals, est, tab, out,
         acc, gbuf, colv, colv2, rowv, valv, esv, gsem, msem) = refs
        xprev = None
    cid = lax.axis_index("c")
    sid = lax.axis_index("s")
    wid = sid * 2 + cid
    r0 = wid * RPW
    pltpu.sync_copy(est, esv)
    widv = jnp.full((LANES,), wid, jnp.int32)
    e0 = plsc.load_gather(esv, [widv])[0]
    e1 = plsc.load_gather(esv, [widv + 1])[0]
    e0a = (e0 >> 4) << 4
    nb = (e1 - e0a + (GK - 1)) // GK

    def chunk_body(ch, _):
        rowbase = ch * NPAD + r0
        if has_prev:
            pltpu.sync_copy(xprev.at[pl.ds(rowbase, RPW)], acc)

            def init_body(i, _):
                for k in range(FC // LANES):
                    sl = pl.ds(k * LANES, LANES)
                    acc[i, sl] = -acc[i, sl]
                return 0
        else:
            zv = jnp.zeros((LANES,), jnp.float32)

            def init_body(i, _):
                for k in range(FC // LANES):
                    acc[i, pl.ds(k * LANES, LANES)] = zv
                return 0
        lax.fori_loop(0, RPW, init_body, 0)

        choff = ch * NPAD

        def ebase(g):
            return pl.multiple_of(e0a + g * GK, 16)

        def meta_issue(g):
            base, slot = ebase(g), lax.rem(g, 3)
            pltpu.async_copy(cols.at[pl.ds(base, GK)], colv.at[slot], msem)
            pltpu.async_copy(rows.at[pl.ds(base, GK)], rowv.at[slot], msem)
            pltpu.async_copy(vals.at[pl.ds(base, GK)], valv.at[slot], msem)

        def meta_wait(g):
            base, slot = ebase(g), lax.rem(g, 3)
            pltpu.make_async_copy(
                cols.at[pl.ds(base, GK)], colv.at[slot], msem).wait()
            pltpu.make_async_copy(
                rows.at[pl.ds(base, GK)], rowv.at[slot], msem).wait()
            pltpu.make_async_copy(
                vals.at[pl.ds(base, GK)], valv.at[slot], msem).wait()

        def gather_issue(g):
            slot, mslot = g & 1, lax.rem(g, 3)
            for t in range(GK // LANES):
                sl = pl.ds(t * LANES, LANES)
                colv2[slot, sl] = colv[mslot, sl] + choff
            pltpu.async_copy(tab.at[colv2.at[slot]], gbuf.at[slot], gsem)

        def gather_wait(g):
            slot = g & 1
            pltpu.make_async_copy(
                tab.at[colv2.at[slot]], gbuf.at[slot], gsem).wait()

        @pl.when(nb > 0)
        def _():
            meta_issue(0)
            meta_wait(0)
            gather_issue(0)

            @pl.when(nb > 1)
            def _():
                meta_issue(1)

        lane = lax.iota(jnp.int32, LANES)

        def batch_body(g, _):
            @pl.when(g + 1 < nb)
            def _():
                meta_wait(g + 1)
                gather_issue(g + 1)

            @pl.when(g + 2 < nb)
            def _():
                meta_issue(g + 2)

            gather_wait(g)
            base, slot, mslot = ebase(g), g & 1, lax.rem(g, 3)
            rlocs, vvs = [], []
            for t in range(GK // LANES):
                sl = pl.ds(t * LANES, LANES)
                eiv = base + t * LANES + lane
                validv = (eiv >= e0) & (eiv < e1)
                rlocv = jnp.where(validv, rowv[mslot, sl] - r0, 0)
                vvv = jnp.where(validv, valv[mslot, sl], 0.0) * alpha
                for j in range(LANES):
                    rlocs.append(rlocv[j])
                    vvs.append(vvv[j])

            def loads(jj):
                return [gbuf[slot, jj, pl.ds(k * LANES, LANES)] * vvs[jj]
                        for k in range(FC // LANES)]

            gs_cur = loads(0)
            for j in range(GK):
                gs_next = loads(j + 1) if j + 1 < GK else None
                for k in range(FC // LANES):
                    slk = pl.ds(k * LANES, LANES)
                    plsc.addupdate(acc.at[rlocs[j], slk], gs_cur[k])
                gs_cur = gs_next
            return 0

        lax.fori_loop(0, nb, batch_body, 0)
        pltpu.sync_copy(acc, out.at[pl.ds(rowbase, RPW)])
        return 0

    lax.fori_loop(0, NCH, chunk_body, 0)


@functools.lru_cache(maxsize=None)
def _make_spmm(has_prev):
    mesh = plsc.VectorSubcoreMesh(core_axis_name="c", subcore_axis_name="s")
    out_type = jax.ShapeDtypeStruct((NCH * NPAD, FC), jnp.float32)
    scratch = [
        pltpu.VMEM((RPW, FC), jnp.float32),
        pltpu.VMEM((2, GK, FC), jnp.float32),
        pltpu.VMEM((3, GK), jnp.int32),
        pltpu.VMEM((2, GK), jnp.int32),
        pltpu.VMEM((3, GK), jnp.int32),
        pltpu.VMEM((3, GK), jnp.float32),
        pltpu.VMEM((40,), jnp.int32),
        pltpu.SemaphoreType.DMA,
        pltpu.SemaphoreType.DMA,
    ]
    alpha = 2.0 if has_prev else 1.0
    body = functools.partial(_spmm_body, has_prev, alpha)
    return pl.kernel(body, out_type=out_type, mesh=mesh,
                     scratch_types=scratch,
                     compiler_params=pltpu.CompilerParams(
                         needs_layout_passes=False))


def _mm_body(x_ref, w_ref, b_ref, o_ref):
    o_ref[:] = jnp.dot(x_ref[:], w_ref[:],
                       preferred_element_type=jnp.float32) + b_ref[:]


def _matmul(x, w, b):
    bm = 2000
    m = x.shape[0]
    return pl.pallas_call(
        _mm_body,
        grid=(m // bm,),
        in_specs=[
            pl.BlockSpec((bm, x.shape[1]), lambda i: (i, 0)),
            pl.BlockSpec(w.shape, lambda i: (0, 0)),
            pl.BlockSpec((1, OUTPUT_DIM), lambda i: (0, 0)),
        ],
        out_specs=pl.BlockSpec((bm, OUTPUT_DIM), lambda i: (i, 0)),
        out_shape=jax.ShapeDtypeStruct((m, OUTPUT_DIM), jnp.float32),
    )(x, w, b)


def kernel(inputs, state, sup0_rows, sup0_cols, sup0_vals,
           sup1_rows, sup1_cols, sup1_vals, weight, biases):
    inp = inputs.reshape(B, N, INPUT_DIM)
    st = state.reshape(B, N, HID_DIM)
    x = jnp.concatenate([inp, st], axis=2)
    x0 = jnp.transpose(x, (1, 2, 0)).reshape(N, C)
    x0p = jnp.pad(x0, ((0, NPAD - N), (0, 0)))
    x0c = x0p.reshape(NPAD, NCH, FC).transpose(1, 0, 2).reshape(NCH * NPAD, FC)

    spmm0 = _make_spmm(False)
    spmm1 = _make_spmm(True)
    bounds = jnp.arange(NW + 1, dtype=jnp.int32) * RPW
    outs = [x0c]
    for rows, cols, vals in ((sup0_rows, sup0_cols, sup0_vals),
                             (sup1_rows, sup1_cols, sup1_vals)):
        est = jnp.searchsorted(rows, bounds).astype(jnp.int32)
        est = jnp.pad(est, (0, 40 - (NW + 1)))
        rp = jnp.pad(rows, (0, GK))
        cp = jnp.pad(cols, (0, GK))
        vp = jnp.pad(vals, (0, GK))
        x1 = spmm0(rp, cp, vp, est, x0c)
        x2 = spmm1(rp, cp, vp, est, x1, x0c)
        outs.extend([x1, x2])

    xs = jnp.stack(outs)
    xs = xs.reshape(NUM_MATRICES, NCH, NPAD, FC)[:, :, :N]
    xs = xs.reshape(NUM_MATRICES, NCH, N, FC // B, B)
    xcat = jnp.transpose(xs, (4, 2, 0, 1, 3)).reshape(
        B * N, INPUT_SIZE * NUM_MATRICES)
    wperm = weight[jnp.asarray(_PERM)]
    out = _matmul(xcat, wperm, biases.reshape(1, OUTPUT_DIM))
    return out.reshape(B, N * OUTPUT_DIM)


# speedup vs baseline: 1.0275x; 1.0275x over previous
import functools

import numpy as np
import jax
import jax.numpy as jnp
from jax import lax
from jax.experimental import pallas as pl
from jax.experimental.pallas import tpu as pltpu
from jax.experimental.pallas import tpu_sc as plsc

N = 10000
E = 320000
INPUT_DIM = 64
HID_DIM = 64
OUTPUT_DIM = 64
B = 8
INPUT_SIZE = INPUT_DIM + HID_DIM
NUM_MATRICES = 5
C = INPUT_SIZE * B

NW = 32
RPW = 320
NPAD = NW * RPW
NCH = 4
FC = C // NCH
GK = 64
LANES = 16

_idx = np.arange(INPUT_SIZE * NUM_MATRICES)
_PERM = np.asarray((_idx % INPUT_SIZE) * NUM_MATRICES + _idx // INPUT_SIZE,
                   dtype=np.int32)


def _spmm_body(has_prev, alpha, *refs):
    if has_prev:
        (rows, cols, vals, est, tab, xprev, out,
         acc, gbuf, colv, colv2, rowv, valv, esv, gsem, msem) = refs
    else:
        (rows, cols, vals, est, tab, out,
         acc, gbuf, colv, colv2, rowv, valv, esv, gsem, msem) = refs
        xprev = None
    cid = lax.axis_index("c")
    sid = lax.axis_index("s")
    wid = sid * 2 + cid
    r0 = wid * RPW
    pltpu.sync_copy(est, esv)
    widv = jnp.full((LANES,), wid, jnp.int32)
    e0 = plsc.load_gather(esv, [widv])[0]
    e1 = plsc.load_gather(esv, [widv + 1])[0]
    e0a = (e0 >> 4) << 4
    nb = (e1 - e0a + (GK - 1)) // GK

    def chunk_body(ch, _):
        rowbase = ch * NPAD + r0
        if has_prev:
            pltpu.sync_copy(xprev.at[pl.ds(rowbase, RPW)], acc)

            def init_body(i, _):
                for k in range(FC // LANES):
                    sl = pl.ds(k * LANES, LANES)
                    acc[i, sl] = -acc[i, sl]
                return 0
        else:
            zv = jnp.zeros((LANES,), jnp.float32)

            def init_body(i, _):
                for k in range(FC // LANES):
                    acc[i, pl.ds(k * LANES, LANES)] = zv
                return 0
        lax.fori_loop(0, RPW, init_body, 0)

        choff = ch * NPAD

        def ebase(g):
            return pl.multiple_of(e0a + g * GK, 16)

        def meta_issue(g):
            base, slot = ebase(g), lax.rem(g, 3)
            pltpu.async_copy(cols.at[pl.ds(base, GK)], colv.at[slot], msem)
            pltpu.async_copy(rows.at[pl.ds(base, GK)], rowv.at[slot], msem)
            pltpu.async_copy(vals.at[pl.ds(base, GK)], valv.at[slot], msem)

        def meta_wait(g):
            base, slot = ebase(g), lax.rem(g, 3)
            pltpu.make_async_copy(
                cols.at[pl.ds(base, GK)], colv.at[slot], msem).wait()
            pltpu.make_async_copy(
                rows.at[pl.ds(base, GK)], rowv.at[slot], msem).wait()
            pltpu.make_async_copy(
                vals.at[pl.ds(base, GK)], valv.at[slot], msem).wait()

        def gather_issue(g):
            slot, mslot = g & 1, lax.rem(g, 3)
            for t in range(GK // LANES):
                sl = pl.ds(t * LANES, LANES)
                colv2[slot, sl] = colv[mslot, sl] + choff
            pltpu.async_copy(tab.at[colv2.at[slot]], gbuf.at[slot], gsem)

        def gather_wait(g):
            slot = g & 1
            pltpu.make_async_copy(
                tab.at[colv2.at[slot]], gbuf.at[slot], gsem).wait()

        @pl.when(nb > 0)
        def _():
            meta_issue(0)
            meta_wait(0)
            gather_issue(0)

            @pl.when(nb > 1)
            def _():
                meta_issue(1)

        lane = lax.iota(jnp.int32, LANES)

        def batch_body(g, _):
            @pl.when(g + 1 < nb)
            def _():
                meta_wait(g + 1)
                gather_issue(g + 1)

            @pl.when(g + 2 < nb)
            def _():
                meta_issue(g + 2)

            gather_wait(g)
            base, slot, mslot = ebase(g), g & 1, lax.rem(g, 3)
            for t in range(GK // LANES):
                sl = pl.ds(t * LANES, LANES)
                eiv = base + t * LANES + lane
                validv = (eiv >= e0) & (eiv < e1)
                rlocv = jnp.where(validv, rowv[mslot, sl] - r0, 0)
                vvv = jnp.where(validv, valv[mslot, sl], 0.0) * alpha

                def loads(j):
                    jj = t * LANES + j
                    vv = vvv[j]
                    return [gbuf[slot, jj, pl.ds(k * LANES, LANES)] * vv
                            for k in range(FC // LANES)]

                gs_cur = loads(0)
                for j in range(LANES):
                    gs_next = loads(j + 1) if j + 1 < LANES else None
                    rloc = rlocv[j]
                    for k in range(FC // LANES):
                        slk = pl.ds(k * LANES, LANES)
                        plsc.addupdate(acc.at[rloc, slk], gs_cur[k])
                    gs_cur = gs_next
            return 0

        lax.fori_loop(0, nb, batch_body, 0)
        pltpu.sync_copy(acc, out.at[pl.ds(rowbase, RPW)])
        return 0

    lax.fori_loop(0, NCH, chunk_body, 0)


@functools.lru_cache(maxsize=None)
def _make_spmm(has_prev):
    mesh = plsc.VectorSubcoreMesh(core_axis_name="c", subcore_axis_name="s")
    out_type = jax.ShapeDtypeStruct((NCH * NPAD, FC), jnp.float32)
    scratch = [
        pltpu.VMEM((RPW, FC), jnp.float32),
        pltpu.VMEM((2, GK, FC), jnp.float32),
        pltpu.VMEM((3, GK), jnp.int32),
        pltpu.VMEM((2, GK), jnp.int32),
        pltpu.VMEM((3, GK), jnp.int32),
        pltpu.VMEM((3, GK), jnp.float32),
        pltpu.VMEM((40,), jnp.int32),
        pltpu.SemaphoreType.DMA,
        pltpu.SemaphoreType.DMA,
    ]
    alpha = 2.0 if has_prev else 1.0
    body = functools.partial(_spmm_body, has_prev, alpha)
    return pl.kernel(body, out_type=out_type, mesh=mesh,
                     scratch_types=scratch,
                     compiler_params=pltpu.CompilerParams(
                         needs_layout_passes=False))


def _mm_body(x_ref, w_ref, b_ref, o_ref):
    o_ref[:] = jnp.dot(x_ref[:], w_ref[:],
                       preferred_element_type=jnp.float32) + b_ref[:]


def _matmul(x, w, b):
    bm = 2000
    m = x.shape[0]
    return pl.pallas_call(
        _mm_body,
        grid=(m // bm,),
        in_specs=[
            pl.BlockSpec((bm, x.shape[1]), lambda i: (i, 0)),
            pl.BlockSpec(w.shape, lambda i: (0, 0)),
            pl.BlockSpec((1, OUTPUT_DIM), lambda i: (0, 0)),
        ],
        out_specs=pl.BlockSpec((bm, OUTPUT_DIM), lambda i: (i, 0)),
        out_shape=jax.ShapeDtypeStruct((m, OUTPUT_DIM), jnp.float32),
    )(x, w, b)


def kernel(inputs, state, sup0_rows, sup0_cols, sup0_vals,
           sup1_rows, sup1_cols, sup1_vals, weight, biases):
    inp = inputs.reshape(B, N, INPUT_DIM)
    st = state.reshape(B, N, HID_DIM)
    x = jnp.concatenate([inp, st], axis=2)
    x0 = jnp.transpose(x, (1, 2, 0)).reshape(N, C)
    x0p = jnp.pad(x0, ((0, NPAD - N), (0, 0)))
    x0c = x0p.reshape(NPAD, NCH, FC).transpose(1, 0, 2).reshape(NCH * NPAD, FC)

    spmm0 = _make_spmm(False)
    spmm1 = _make_spmm(True)
    bounds = jnp.arange(NW + 1, dtype=jnp.int32) * RPW
    outs = [x0c]
    for rows, cols, vals in ((sup0_rows, sup0_cols, sup0_vals),
                             (sup1_rows, sup1_cols, sup1_vals)):
        est = jnp.searchsorted(rows, bounds).astype(jnp.int32)
        est = jnp.pad(est, (0, 40 - (NW + 1)))
        rp = jnp.pad(rows, (0, GK))
        cp = jnp.pad(cols, (0, GK))
        vp = jnp.pad(vals, (0, GK))
        x1 = spmm0(rp, cp, vp, est, x0c)
        x2 = spmm1(rp, cp, vp, est, x1, x0c)
        outs.extend([x1, x2])

    xs = jnp.stack(outs)
    xs = xs.reshape(NUM_MATRICES, NCH, NPAD, FC)[:, :, :N]
    xs = xs.reshape(NUM_MATRICES, NCH, N, FC // B, B)
    xcat = jnp.transpose(xs, (4, 2, 0, 1, 3)).reshape(
        B * N, INPUT_SIZE * NUM_MATRICES)
    wperm = weight[jnp.asarray(_PERM)]
    out = _matmul(xcat, wperm, biases.reshape(1, OUTPUT_DIM))
    return out.reshape(B, N * OUTPUT_DIM)
